# baseline (speedup 1.0000x reference)
import functools
import math

import jax
import jax.numpy as jnp
import numpy as np
from jax import lax
from jax.experimental import pallas as pl
from jax.experimental.pallas import tpu as pltpu
from jax.experimental.pallas import tpu_sc as plsc

EPS = 1e-5


def _pos_encoding(length, d):
    pe = np.zeros((length, d), dtype=np.float32)
    position = np.arange(0, length, dtype=np.float32)[:, None]
    div_term = np.exp(
        np.arange(0, d, 2, dtype=np.float32) * -(math.log(10000.0) / d))
    pe[:, 0::2] = np.sin(position * div_term)
    pe[:, 1::2] = np.cos(position * div_term)
    return jnp.asarray(pe)


def _sc_gather(seq_rs, table, n, d, nw, nc, nchunk, chunk):
    rows_per_w = nchunk * chunk

    mesh = plsc.VectorSubcoreMesh(core_axis_name="c", subcore_axis_name="s")

    @functools.partial(
        pl.kernel,
        mesh=mesh,
        compiler_params=pltpu.CompilerParams(needs_layout_passes=False),
        out_type=jax.ShapeDtypeStruct((n, d), jnp.float32),
        scratch_types=[
            pltpu.VMEM((nchunk, chunk), jnp.int32),
            pltpu.VMEM((chunk, d), jnp.float32),
            pltpu.VMEM((chunk, d), jnp.float32),
            pltpu.SemaphoreType.DMA,
            pltpu.SemaphoreType.DMA,
            pltpu.SemaphoreType.DMA,
            pltpu.SemaphoreType.DMA,
        ],
    )
    def sc_fn(seq_hbm, table_hbm, out_hbm,
              idx_all, rows_a, rows_b, gsem_a, gsem_b, osem_a, osem_b):
        wid = lax.axis_index("s") * nc + lax.axis_index("c")
        pltpu.sync_copy(seq_hbm.at[wid], idx_all)
        base = wid * rows_per_w

        rows = (rows_a, rows_b)
        gsem = (gsem_a, gsem_b)
        osem = (osem_a, osem_b)

        def gather_issue(c, p):
            pltpu.async_copy(table_hbm.at[idx_all.at[c]], rows[p], gsem[p])

        def gather_wait(c, p):
            pltpu.make_async_copy(
                table_hbm.at[idx_all.at[c]], rows[p], gsem[p]).wait()

        def out_issue(c, p):
            off = base + c * chunk
            pltpu.async_copy(rows[p], out_hbm.at[pl.ds(off, chunk)], osem[p])

        def out_wait(c, p):
            off = base + c * chunk
            pltpu.make_async_copy(
                rows[p], out_hbm.at[pl.ds(off, chunk)], osem[p]).wait()

        gather_issue(0, 0)
        gather_wait(0, 0)
        gather_issue(1, 1)
        out_issue(0, 0)

        def pair_body(k, carry):
            c1 = 2 * k + 1
            gather_wait(c1, 1)
            out_wait(c1 - 1, 0)
            gather_issue(c1 + 1, 0)
            out_issue(c1, 1)

            c2 = c1 + 1
            gather_wait(c2, 0)
            out_wait(c2 - 1, 1)
            gather_issue(c2 + 1, 1)
            out_issue(c2, 0)
            return carry

        lax.fori_loop(0, (nchunk - 2) // 2, pair_body, 0)

        cl = nchunk - 1
        gather_wait(cl, 1)
        out_wait(cl - 1, 0)
        out_issue(cl, 1)
        out_wait(cl, 1)

    return sc_fn(seq_rs, table)


def _tc_layernorm(rows, pe, gamma, beta, b_sz, seq_len, d, bb):
    def tc_fn(x_ref, pe_ref, g_ref, b_ref, o_ref):
        x = x_ref[...] + pe_ref[...]
        mean = jnp.mean(x, axis=-1, keepdims=True)
        xc = x - mean
        var = jnp.mean(xc * xc, axis=-1, keepdims=True)
        o_ref[...] = xc * lax.rsqrt(var + EPS) * g_ref[...] + b_ref[...]

    return pl.pallas_call(
        tc_fn,
        grid=(b_sz // bb,),
        in_specs=[
            pl.BlockSpec((bb, seq_len, d), lambda i: (i, 0, 0)),
            pl.BlockSpec((1, seq_len, d), lambda i: (0, 0, 0)),
            pl.BlockSpec((1, 1, d), lambda i: (0, 0, 0)),
            pl.BlockSpec((1, 1, d), lambda i: (0, 0, 0)),
        ],
        out_specs=pl.BlockSpec((bb, seq_len, d), lambda i: (i, 0, 0)),
        out_shape=jax.ShapeDtypeStruct((b_sz, seq_len, d), jnp.float32),
    )(rows, pe, gamma, beta)


def kernel(sequence, table, gamma, beta):
    b_sz, seq_len = sequence.shape
    _, d = table.shape
    n = b_sz * seq_len

    info = plsc.get_sparse_core_info()
    nc, ns = info.num_cores, info.num_subcores
    nw = nc * ns

    pe = _pos_encoding(seq_len, d).reshape(1, seq_len, d)
    g3 = gamma.reshape(1, 1, d)
    b3 = beta.reshape(1, 1, d)

    nslice = 4
    chunk = 80
    bs = b_sz // nslice
    ns_rows = bs * seq_len
    outs = []
    for si in range(nslice):
        seq_sl = lax.slice_in_dim(sequence, si * bs, (si + 1) * bs, axis=0)
        seq_rs = seq_sl.reshape(nw, ns_rows // nw // chunk, chunk).astype(
            jnp.int32)
        rows = _sc_gather(seq_rs, table, ns_rows, d, nw, nc,
                          ns_rows // nw // chunk, chunk)
        outs.append(_tc_layernorm(rows.reshape(bs, seq_len, d), pe, g3, b3,
                                  bs, seq_len, d, bb=32))
    return jnp.concatenate(outs, axis=0)


# speedup vs baseline: 1.4074x; 1.4074x over previous
import functools
import math

import jax
import jax.numpy as jnp
import numpy as np
from jax import lax
from jax.experimental import pallas as pl
from jax.experimental.pallas import tpu as pltpu
from jax.experimental.pallas import tpu_sc as plsc

EPS = 1e-5


def _pos_encoding(length, d):
    pe = np.zeros((length, d), dtype=np.float32)
    position = np.arange(0, length, dtype=np.float32)[:, None]
    div_term = np.exp(
        np.arange(0, d, 2, dtype=np.float32) * -(math.log(10000.0) / d))
    pe[:, 0::2] = np.sin(position * div_term)
    pe[:, 1::2] = np.cos(position * div_term)
    return jnp.asarray(pe)


def _sc_gather(seq_rs, table, n, d, nw, nc, nchunk, chunk):
    rows_per_w = nchunk * chunk

    mesh = plsc.VectorSubcoreMesh(core_axis_name="c", subcore_axis_name="s")

    @functools.partial(
        pl.kernel,
        mesh=mesh,
        compiler_params=pltpu.CompilerParams(needs_layout_passes=False),
        out_type=jax.ShapeDtypeStruct((n, d), jnp.float32),
        scratch_types=[
            pltpu.VMEM((nchunk, chunk), jnp.int32),
            pltpu.VMEM((chunk, d), jnp.float32),
            pltpu.VMEM((chunk, d), jnp.float32),
            pltpu.SemaphoreType.DMA,
            pltpu.SemaphoreType.DMA,
            pltpu.SemaphoreType.DMA,
            pltpu.SemaphoreType.DMA,
        ],
    )
    def sc_fn(seq_hbm, table_hbm, out_hbm,
              idx_all, rows_a, rows_b, gsem_a, gsem_b, osem_a, osem_b):
        wid = lax.axis_index("s") * nc + lax.axis_index("c")
        pltpu.sync_copy(seq_hbm.at[wid], idx_all)
        base = wid * rows_per_w

        rows = (rows_a, rows_b)
        gsem = (gsem_a, gsem_b)
        osem = (osem_a, osem_b)

        def gather_issue(c, p):
            pltpu.async_copy(table_hbm.at[idx_all.at[c]], rows[p], gsem[p])

        def gather_wait(c, p):
            pltpu.make_async_copy(
                table_hbm.at[idx_all.at[c]], rows[p], gsem[p]).wait()

        def out_issue(c, p):
            off = base + c * chunk
            pltpu.async_copy(rows[p], out_hbm.at[pl.ds(off, chunk)], osem[p])

        def out_wait(c, p):
            off = base + c * chunk
            pltpu.make_async_copy(
                rows[p], out_hbm.at[pl.ds(off, chunk)], osem[p]).wait()

        gather_issue(0, 0)
        gather_wait(0, 0)
        gather_issue(1, 1)
        out_issue(0, 0)

        def pair_body(k, carry):
            c1 = 2 * k + 1
            gather_wait(c1, 1)
            out_wait(c1 - 1, 0)
            gather_issue(c1 + 1, 0)
            out_issue(c1, 1)

            c2 = c1 + 1
            gather_wait(c2, 0)
            out_wait(c2 - 1, 1)
            gather_issue(c2 + 1, 1)
            out_issue(c2, 0)
            return carry

        lax.fori_loop(0, (nchunk - 2) // 2, pair_body, 0)

        cl = nchunk - 1
        gather_wait(cl, 1)
        out_wait(cl - 1, 0)
        out_issue(cl, 1)
        out_wait(cl, 1)

    return sc_fn(seq_rs, table)


def _tc_layernorm(rows, pe, gamma, beta, b_sz, seq_len, d, bb):
    def tc_fn(x_ref, pe_ref, g_ref, b_ref, o_ref):
        x = x_ref[...] + pe_ref[...]
        mean = jnp.mean(x, axis=-1, keepdims=True)
        xc = x - mean
        var = jnp.mean(xc * xc, axis=-1, keepdims=True)
        o_ref[...] = xc * lax.rsqrt(var + EPS) * g_ref[...] + b_ref[...]

    return pl.pallas_call(
        tc_fn,
        grid=(b_sz // bb,),
        in_specs=[
            pl.BlockSpec((bb, seq_len, d), lambda i: (i, 0, 0)),
            pl.BlockSpec((1, seq_len, d), lambda i: (0, 0, 0)),
            pl.BlockSpec((1, 1, d), lambda i: (0, 0, 0)),
            pl.BlockSpec((1, 1, d), lambda i: (0, 0, 0)),
        ],
        out_specs=pl.BlockSpec((bb, seq_len, d), lambda i: (i, 0, 0)),
        out_shape=jax.ShapeDtypeStruct((b_sz, seq_len, d), jnp.float32),
    )(rows, pe, gamma, beta)


def kernel(sequence, table, gamma, beta):
    b_sz, seq_len = sequence.shape
    _, d = table.shape
    n = b_sz * seq_len

    info = plsc.get_sparse_core_info()
    nc, ns = info.num_cores, info.num_subcores
    nw = nc * ns

    pe = _pos_encoding(seq_len, d).reshape(1, seq_len, d)
    g3 = gamma.reshape(1, 1, d)
    b3 = beta.reshape(1, 1, d)

    chunk = 128
    nchunk = n // nw // chunk
    seq_rs = sequence.reshape(nw, nchunk, chunk).astype(jnp.int32)
    rows = _sc_gather(seq_rs, table, n, d, nw, nc, nchunk, chunk)
    return _tc_layernorm(rows.reshape(b_sz, seq_len, d), pe, g3, b3,
                         b_sz, seq_len, d, bb=64)
